# Initial kernel scaffold; baseline (speedup 1.0000x reference)
import jax
import jax.numpy as jnp
from jax.experimental import pallas as pl


def kernel(x, edge_index, W):
    raise NotImplementedError("write your pallas kernel here")


# speedup vs baseline: 8.7438x; 8.7438x over previous
import functools

import jax
import jax.numpy as jnp
from jax import lax
from jax.experimental import pallas as pl
from jax.experimental.pallas import tpu as pltpu
from jax.experimental.pallas import tpu_sc as plsc

NC = 2
NS = 16
NW = NC * NS
B = 125


def _sc_segment_sum(x, src2d, dst2d, n_pad):
    n_nodes, d = x.shape
    n_edges = src2d.shape[0] * src2d.shape[1]
    ept = n_edges // NW
    nchunk = ept // B
    rows_pt = n_pad // NS
    rchunk = 64
    nrc = rows_pt // rchunk
    assert ept % B == 0 and rows_pt % rchunk == 0 and d % 16 == 0
    assert nchunk % 8 == 0 and rows_pt % 8 == 0 and B >= rchunk

    mesh = plsc.VectorSubcoreMesh(core_axis_name="c", subcore_axis_name="s")

    @functools.partial(
        pl.kernel,
        mesh=mesh,
        out_type=jax.ShapeDtypeStruct((NC, n_pad, d), jnp.float32),
        scratch_types=[
            pltpu.VMEM((nchunk, B), jnp.int32),
            pltpu.VMEM((nchunk, B), jnp.int32),
            pltpu.VMEM((B, d), jnp.float32),
            pltpu.VMEM_SHARED((n_pad, d), jnp.float32),
            pltpu.SemaphoreType.DMA,
        ],
    )
    def k(x_hbm, src_hbm, dst_hbm, out_hbm, src_v, dst_v, rows_v, acc, sem):
        bounce = rows_v.at[pl.ds(0, rchunk)]
        c = lax.axis_index("c")
        s = lax.axis_index("s")
        wid = s * NC + c

        pltpu.sync_copy(src_hbm.at[pl.ds(wid * nchunk, nchunk)], src_v)
        pltpu.sync_copy(dst_hbm.at[pl.ds(wid * nchunk, nchunk)], dst_v)

        zeros = jnp.zeros((16,), jnp.float32)

        def zrow(i, carry):
            def zcol(j, carry2):
                rows_v[i, pl.ds(j * 16, 16)] = zeros
                return carry2
            return lax.fori_loop(0, d // 16, zcol, carry)

        lax.fori_loop(0, rchunk, zrow, 0)
        row0 = s * rows_pt
        for r in range(nrc):
            pltpu.sync_copy(bounce, acc.at[pl.ds(row0 + r * rchunk, rchunk)])
        plsc.subcore_barrier()

        def body(j, carry):
            pltpu.async_copy(x_hbm.at[src_v.at[j]], rows_v, sem).wait()
            pltpu.sync_copy(rows_v, acc.at[dst_v.at[j]], add=True)
            return carry

        lax.fori_loop(0, nchunk, body, 0)
        plsc.subcore_barrier()

        for r in range(nrc):
            pltpu.sync_copy(acc.at[pl.ds(row0 + r * rchunk, rchunk)], bounce)
            pltpu.sync_copy(bounce, out_hbm.at[c, pl.ds(row0 + r * rchunk, rchunk)])

    return k(x, src2d, dst2d)


def _tc_transform(partials, w, n_nodes):
    d = partials.shape[2]
    r = 1000
    grid = (n_nodes // r,)

    def body(p_ref, w_ref, o_ref):
        o_ref[...] = jnp.dot(p_ref[0] + p_ref[1], w_ref[...],
                             preferred_element_type=jnp.float32)

    return pl.pallas_call(
        body,
        grid=grid,
        in_specs=[
            pl.BlockSpec((2, r, d), lambda i: (0, i, 0)),
            pl.BlockSpec((d, d), lambda i: (0, 0)),
        ],
        out_specs=pl.BlockSpec((r, d), lambda i: (i, 0)),
        out_shape=jax.ShapeDtypeStruct((n_nodes, d), jnp.float32),
    )(partials, w)


def kernel(x, edge_index, W):
    n_nodes = x.shape[0]
    n_edges = edge_index.shape[1]
    n_pad = ((n_nodes + NS * 128 - 1) // (NS * 128)) * (NS * 128)
    src = edge_index[0].astype(jnp.int32).reshape(n_edges // B, B)
    dst = edge_index[1].astype(jnp.int32).reshape(n_edges // B, B)
    partials = _sc_segment_sum(x, src, dst, n_pad)
    return _tc_transform(partials, W, n_nodes)


# speedup vs baseline: 12.5829x; 1.4391x over previous
import functools

import jax
import jax.numpy as jnp
from jax import lax
from jax.experimental import pallas as pl
from jax.experimental.pallas import tpu as pltpu
from jax.experimental.pallas import tpu_sc as plsc

NC = 2
NS = 16
NW = NC * NS
B = 125


def _sc_segment_sum(x, src3d, dst3d, n_pad):
    n_nodes, d = x.shape
    nchunk = src3d.shape[1]
    nhalf = nchunk // 2
    rows_pt = n_pad // NS
    rchunk = 40
    nrc = rows_pt // rchunk
    assert nhalf % 2 == 0 and rows_pt % rchunk == 0 and d % 16 == 0
    assert B >= rchunk and rchunk % 8 == 0 and nhalf % 8 == 0

    mesh = plsc.VectorSubcoreMesh(core_axis_name="c", subcore_axis_name="s")

    @functools.partial(
        pl.kernel,
        mesh=mesh,
        out_type=jax.ShapeDtypeStruct((NC, n_pad, d), jnp.float32),
        scratch_types=[
            pltpu.VMEM((nhalf, B), jnp.int32),
            pltpu.VMEM((nhalf, B), jnp.int32),
            pltpu.VMEM((B, d), jnp.float32),
            pltpu.VMEM((B, d), jnp.float32),
            pltpu.VMEM_SHARED((n_pad, d), jnp.float32),
            pltpu.SemaphoreType.DMA,
            pltpu.SemaphoreType.DMA,
        ],
    )
    def k(x_hbm, src_hbm, dst_hbm, out_hbm,
          src_v, dst_v, rows_a, rows_b, acc, sem_a, sem_b):
        c = lax.axis_index("c")
        s = lax.axis_index("s")
        wid = s * NC + c
        bounce = rows_a.at[pl.ds(0, rchunk)]

        def stage_indices(j0, sync):
            cp_s = pltpu.async_copy(src_hbm.at[wid, pl.ds(j0, nhalf)], src_v, sem_a)
            cp_d = pltpu.async_copy(dst_hbm.at[wid, pl.ds(j0, nhalf)], dst_v, sem_b)
            if sync:
                cp_s.wait()
                cp_d.wait()
            return cp_s, cp_d

        cp_s, cp_d = stage_indices(0, sync=False)

        zeros = jnp.zeros((16,), jnp.float32)

        def zrow(i, carry):
            def zcol(j, carry2):
                rows_a[i, pl.ds(j * 16, 16)] = zeros
                return carry2
            return lax.fori_loop(0, d // 16, zcol, carry)

        lax.fori_loop(0, rchunk, zrow, 0)
        row0 = s * rows_pt
        for r in range(nrc):
            pltpu.sync_copy(bounce, acc.at[pl.ds(row0 + r * rchunk, rchunk)])
        cp_s.wait()
        cp_d.wait()
        plsc.subcore_barrier()

        def gather(j, buf, sem):
            pltpu.async_copy(x_hbm.at[src_v.at[j]], buf, sem)

        def wait_gather(j, buf, sem):
            pltpu.make_async_copy(x_hbm.at[src_v.at[j]], buf, sem).wait()

        def scatter(j, buf):
            pltpu.sync_copy(buf, acc.at[dst_v.at[j]], add=True)

        def run_stage():
            gather(0, rows_a, sem_a)
            gather(1, rows_b, sem_b)

            def body(m, carry):
                j = 2 * m
                wait_gather(j, rows_a, sem_a)
                scatter(j, rows_a)
                gather(j + 2, rows_a, sem_a)
                wait_gather(j + 1, rows_b, sem_b)
                scatter(j + 1, rows_b)
                gather(j + 3, rows_b, sem_b)
                return carry

            lax.fori_loop(0, nhalf // 2 - 1, body, 0)
            j = nhalf - 2
            wait_gather(j, rows_a, sem_a)
            scatter(j, rows_a)
            wait_gather(j + 1, rows_b, sem_b)
            scatter(j + 1, rows_b)

        run_stage()
        stage_indices(nhalf, sync=True)
        run_stage()
        plsc.subcore_barrier()

        for r in range(nrc):
            pltpu.sync_copy(acc.at[pl.ds(row0 + r * rchunk, rchunk)], bounce)
            pltpu.sync_copy(bounce, out_hbm.at[c, pl.ds(row0 + r * rchunk, rchunk)])

    return k(x, src3d, dst3d)


def _tc_transform(partials, w, n_nodes):
    d = partials.shape[2]
    r = 1000
    grid = (n_nodes // r,)

    def body(p_ref, w_ref, o_ref):
        o_ref[...] = jnp.dot(p_ref[0] + p_ref[1], w_ref[...],
                             preferred_element_type=jnp.float32)

    return pl.pallas_call(
        body,
        grid=grid,
        in_specs=[
            pl.BlockSpec((2, r, d), lambda i: (0, i, 0)),
            pl.BlockSpec((d, d), lambda i: (0, 0)),
        ],
        out_specs=pl.BlockSpec((r, d), lambda i: (i, 0)),
        out_shape=jax.ShapeDtypeStruct((n_nodes, d), jnp.float32),
    )(partials, w)


def kernel(x, edge_index, W):
    n_nodes = x.shape[0]
    n_edges = edge_index.shape[1]
    n_pad = ((n_nodes + NS * 128 - 1) // (NS * 128)) * (NS * 128)
    ept = n_edges // NW
    assert ept % B == 0
    src = edge_index[0].astype(jnp.int32).reshape(NW, ept // B, B)
    dst = edge_index[1].astype(jnp.int32).reshape(NW, ept // B, B)
    partials = _sc_segment_sum(x, src, dst, n_pad)
    return _tc_transform(partials, W, n_nodes)


# speedup vs baseline: 13.7945x; 1.0963x over previous
import functools

import jax
import jax.numpy as jnp
from jax import lax
from jax.experimental import pallas as pl
from jax.experimental.pallas import tpu as pltpu
from jax.experimental.pallas import tpu_sc as plsc

NC = 2
NS = 16
NW = NC * NS
B = 125


def _sc_segment_sum(x, edges4d, n_pad):
    n_nodes, d = x.shape
    nchunk = edges4d.shape[2]
    nhalf = nchunk // 2
    rows_pt = n_pad // NS
    rchunk = 40
    nrc = rows_pt // rchunk
    assert nhalf % 2 == 0 and rows_pt % rchunk == 0 and d % 16 == 0
    assert B >= rchunk and rchunk % 8 == 0 and nhalf % 8 == 0

    mesh = plsc.VectorSubcoreMesh(core_axis_name="c", subcore_axis_name="s")

    @functools.partial(
        pl.kernel,
        mesh=mesh,
        out_type=jax.ShapeDtypeStruct((NC, n_pad, d), jnp.float32),
        scratch_types=[
            pltpu.VMEM((nhalf, B), jnp.int32),
            pltpu.VMEM((nhalf, B), jnp.int32),
            pltpu.VMEM((B, d), jnp.float32),
            pltpu.VMEM((B, d), jnp.float32),
            pltpu.VMEM_SHARED((n_pad, d), jnp.float32),
            pltpu.SemaphoreType.DMA,
            pltpu.SemaphoreType.DMA,
        ],
    )
    def k(x_hbm, edges_hbm, out_hbm,
          src_v, dst_v, rows_a, rows_b, acc, sem_a, sem_b):
        c = lax.axis_index("c")
        s = lax.axis_index("s")
        wid = s * NC + c
        bounce = rows_a.at[pl.ds(0, rchunk)]

        def stage_indices(j0, sync):
            cp_s = pltpu.async_copy(edges_hbm.at[0, wid, pl.ds(j0, nhalf)],
                                    src_v, sem_a)
            cp_d = pltpu.async_copy(edges_hbm.at[1, wid, pl.ds(j0, nhalf)],
                                    dst_v, sem_b)
            if sync:
                cp_s.wait()
                cp_d.wait()
            return cp_s, cp_d

        cp_s, cp_d = stage_indices(0, sync=False)

        zeros = jnp.zeros((16,), jnp.float32)

        def zrow(i, carry):
            def zcol(j, carry2):
                rows_a[i, pl.ds(j * 16, 16)] = zeros
                return carry2
            return lax.fori_loop(0, d // 16, zcol, carry)

        lax.fori_loop(0, rchunk, zrow, 0)
        row0 = s * rows_pt
        for r in range(nrc):
            pltpu.sync_copy(bounce, acc.at[pl.ds(row0 + r * rchunk, rchunk)])
        cp_s.wait()
        cp_d.wait()
        plsc.subcore_barrier()

        def gather(j, buf, sem):
            pltpu.async_copy(x_hbm.at[src_v.at[j]], buf, sem)

        def wait_gather(j, buf, sem):
            pltpu.make_async_copy(x_hbm.at[src_v.at[j]], buf, sem).wait()

        def scatter(j, buf):
            pltpu.sync_copy(buf, acc.at[dst_v.at[j]], add=True)

        def run_stage():
            gather(0, rows_a, sem_a)
            gather(1, rows_b, sem_b)

            def body(m, carry):
                j = 2 * m
                wait_gather(j, rows_a, sem_a)
                scatter(j, rows_a)
                gather(j + 2, rows_a, sem_a)
                wait_gather(j + 1, rows_b, sem_b)
                scatter(j + 1, rows_b)
                gather(j + 3, rows_b, sem_b)
                return carry

            lax.fori_loop(0, nhalf // 2 - 1, body, 0)
            j = nhalf - 2
            wait_gather(j, rows_a, sem_a)
            scatter(j, rows_a)
            wait_gather(j + 1, rows_b, sem_b)
            scatter(j + 1, rows_b)

        run_stage()
        stage_indices(nhalf, sync=True)
        run_stage()
        plsc.subcore_barrier()

        pltpu.sync_copy(acc.at[pl.ds(row0, rows_pt)],
                        out_hbm.at[c, pl.ds(row0, rows_pt)])

    return k(x, edges4d)


def _tc_transform(partials, w, n_nodes):
    d = partials.shape[2]
    r = 1000
    grid = (n_nodes // r,)

    def body(p_ref, w_ref, o_ref):
        o_ref[...] = jnp.dot(p_ref[0] + p_ref[1], w_ref[...],
                             preferred_element_type=jnp.float32)

    return pl.pallas_call(
        body,
        grid=grid,
        in_specs=[
            pl.BlockSpec((2, r, d), lambda i: (0, i, 0)),
            pl.BlockSpec((d, d), lambda i: (0, 0)),
        ],
        out_specs=pl.BlockSpec((r, d), lambda i: (i, 0)),
        out_shape=jax.ShapeDtypeStruct((n_nodes, d), jnp.float32),
    )(partials, w)


def kernel(x, edge_index, W):
    n_nodes = x.shape[0]
    n_edges = edge_index.shape[1]
    n_pad = ((n_nodes + NS * 128 - 1) // (NS * 128)) * (NS * 128)
    ept = n_edges // NW
    assert ept % B == 0
    edges = edge_index.astype(jnp.int32).reshape(2, NW, ept // B, B)
    partials = _sc_segment_sum(x, edges, n_pad)
    return _tc_transform(partials, W, n_nodes)
